# baseline (speedup 1.0000x reference)
import functools

import jax
import jax.numpy as jnp
from jax import lax
from jax.experimental import pallas as pl
from jax.experimental.pallas import tpu as pltpu
from jax.experimental.pallas import tpu_sc as plsc

NC = 2
NS = 16
NW = NC * NS

VOCAB = 100000
D = 128
DW = D // 2
B = 4096 * 200
NSLICE = 4
BS = B // NSLICE
B_PER_W = BS // NW
CHUNK = 128
NCHUNKS = B_PER_W // CHUNK
NBUF = 5
LOOKAHEAD = 3
LAG = NBUF - LOOKAHEAD

assert NCHUNKS % NBUF == 0 and B % (NSLICE * NW * CHUNK) == 0

_mesh = plsc.VectorSubcoreMesh(core_axis_name="c", subcore_axis_name="s")


@functools.partial(
    pl.kernel,
    out_type=jax.ShapeDtypeStruct((BS, DW), jnp.int32),
    mesh=_mesh,
    compiler_params=pltpu.CompilerParams(use_tc_tiling_on_sc=False),
    scratch_types=[
        pltpu.VMEM((NCHUNKS, CHUNK), jnp.int32),
        pltpu.VMEM((NBUF, CHUNK, DW), jnp.int32),
        [pltpu.SemaphoreType.DMA] * NBUF,
        [pltpu.SemaphoreType.DMA] * NBUF,
    ],
)
def _gather_slice(table_hbm, x_hbm, out_hbm, idx_v, rows_v, gsems, osems):
    wid = lax.axis_index("s") * NC + lax.axis_index("c")
    pltpu.sync_copy(x_hbm.at[wid], idx_v)
    base = wid * B_PER_W

    def gather(j, b):
        return pltpu.make_async_copy(
            table_hbm.at[idx_v.at[j]], rows_v.at[b], gsems[b])

    def outcopy(j, b):
        return pltpu.make_async_copy(
            rows_v.at[b], out_hbm.at[pl.ds(base + j * CHUNK, CHUNK)], osems[b])

    for b in range(LOOKAHEAD):
        gather(b, b).start()

    def ring_body(i, carry):
        j0 = i * NBUF
        for b in range(NBUF):
            j = j0 + b
            gather(j, b).wait()
            outcopy(j, b).start()
            bn = (b + LOOKAHEAD) % NBUF

            @pl.when(j >= LAG)
            def _():
                outcopy(j - LAG, bn).wait()

            @pl.when(j + LOOKAHEAD < NCHUNKS)
            def _():
                gather(j + LOOKAHEAD, bn).start()
        return carry

    lax.fori_loop(0, NCHUNKS // NBUF, ring_body, 0, unroll=False)

    for j in range(NCHUNKS - LAG, NCHUNKS):
        outcopy(j, j % NBUF).wait()


def kernel(x, table):
    tb = lax.bitcast_convert_type(
        table.astype(jnp.bfloat16).reshape(VOCAB, DW, 2), jnp.int32)
    x2d = x.reshape(NSLICE, NW, NCHUNKS, CHUNK).astype(jnp.int32)
    outs = []
    for s in range(NSLICE):
        oi = _gather_slice(tb, x2d[s])
        o16 = lax.bitcast_convert_type(oi, jnp.bfloat16)
        outs.append(o16.reshape(BS, D).astype(jnp.float32))
    return jnp.concatenate(outs, axis=0).reshape(4096, 200, D)


# speedup vs baseline: 8.9977x; 8.9977x over previous
import functools

import jax
import jax.numpy as jnp
from jax import lax
from jax.experimental import pallas as pl
from jax.experimental.pallas import tpu as pltpu
from jax.experimental.pallas import tpu_sc as plsc

NC = 2
NS = 16
NW = NC * NS

VOCAB = 100000
D = 128
B = 4096 * 200
B_PER_W = B // NW
CHUNK = 64
NCHUNKS = B_PER_W // CHUNK
assert NCHUNKS % 1 == 0
NBUF = 8
LOOKAHEAD = 6

_mesh = plsc.VectorSubcoreMesh(core_axis_name="c", subcore_axis_name="s")


@functools.partial(
    pl.kernel,
    out_type=jax.ShapeDtypeStruct((B, D), jnp.float32),
    mesh=_mesh,
    scratch_types=[
        pltpu.VMEM((NCHUNKS, CHUNK), jnp.int32),
        pltpu.VMEM((NBUF, CHUNK, D), jnp.float32),
        [pltpu.SemaphoreType.DMA] * NBUF,
        [pltpu.SemaphoreType.DMA] * NBUF,
    ],
)
def _gather_all(table_hbm, x_hbm, out_hbm, idx_v, rows_v, gsems, osems):
    wid = lax.axis_index("s") * NC + lax.axis_index("c")
    pltpu.sync_copy(x_hbm.at[pl.ds(wid * NCHUNKS, NCHUNKS)], idx_v)
    base = wid * B_PER_W

    def gather(j, b):
        return pltpu.make_async_copy(
            table_hbm.at[idx_v.at[j]], rows_v.at[b], gsems[b])

    def outcopy(j, b):
        return pltpu.make_async_copy(
            rows_v.at[b], out_hbm.at[pl.ds(base + j * CHUNK, CHUNK)], osems[b])

    for b in range(LOOKAHEAD):
        gather(b, b).start()

    LAG = NBUF - LOOKAHEAD

    def ring_body(i, carry):
        j0 = i * NBUF
        for b in range(NBUF):
            j = j0 + b
            gather(j, b).wait()
            outcopy(j, b).start()
            bn = (b + LOOKAHEAD) % NBUF

            @pl.when(j >= LAG)
            def _():
                outcopy(j - LAG, bn).wait()

            @pl.when(j + LOOKAHEAD < NCHUNKS)
            def _():
                gather(j + LOOKAHEAD, bn).start()
        return carry

    lax.fori_loop(0, NCHUNKS // NBUF, ring_body, 0, unroll=False)

    for j in range(NCHUNKS - LAG, NCHUNKS):
        outcopy(j, j % NBUF).wait()


def kernel(x, table):
    x2d = x.reshape(B // CHUNK, CHUNK).astype(jnp.int32)
    out = _gather_all(table, x2d)
    return out.reshape(4096, 200, D)


# speedup vs baseline: 9.0074x; 1.0011x over previous
import functools

import jax
import jax.numpy as jnp
from jax import lax
from jax.experimental import pallas as pl
from jax.experimental.pallas import tpu as pltpu
from jax.experimental.pallas import tpu_sc as plsc

NC = 2
NS = 16
NW = NC * NS

VOCAB = 100000
D = 128
B = 4096 * 200
B_PER_W = B // NW
CHUNK = 64
NCHUNKS = B_PER_W // CHUNK
NBUF = 8
LOOKAHEAD = 6

assert NCHUNKS % NBUF == 0 and B_PER_W % CHUNK == 0 and (NW * CHUNK) % 8 == 0

_mesh = plsc.VectorSubcoreMesh(core_axis_name="c", subcore_axis_name="s")


@functools.partial(
    pl.kernel,
    out_type=jax.ShapeDtypeStruct((B, D), jnp.float32),
    mesh=_mesh,
    scratch_types=[
        pltpu.VMEM((NCHUNKS, CHUNK), jnp.int32),
        pltpu.VMEM((NBUF, CHUNK, D), jnp.float32),
        [pltpu.SemaphoreType.DMA] * NBUF,
        [pltpu.SemaphoreType.DMA] * NBUF,
    ],
)
def _gather_all(table_hbm, x_hbm, out_hbm, idx_v, rows_v, gsems, osems):
    wid = lax.axis_index("s") * NC + lax.axis_index("c")
    pltpu.sync_copy(x_hbm.at[pl.ds(wid * NCHUNKS, NCHUNKS)], idx_v)
    base = wid * B_PER_W

    def gather(j, b):
        return pltpu.make_async_copy(
            table_hbm.at[idx_v.at[j]], rows_v.at[b], gsems[b])

    def outcopy(j, b):
        return pltpu.make_async_copy(
            rows_v.at[b], out_hbm.at[pl.ds(base + j * CHUNK, CHUNK)], osems[b])

    for b in range(LOOKAHEAD):
        gather(b, b).start()

    LAG = NBUF - LOOKAHEAD

    def ring_body(i, carry):
        j0 = i * NBUF
        for b in range(NBUF):
            j = j0 + b
            gather(j, b).wait()
            outcopy(j, b).start()
            bn = (b + LOOKAHEAD) % NBUF

            @pl.when(j >= LAG)
            def _():
                outcopy(j - LAG, bn).wait()

            @pl.when(j + LOOKAHEAD < NCHUNKS)
            def _():
                gather(j + LOOKAHEAD, bn).start()
        return carry

    lax.fori_loop(0, NCHUNKS // NBUF, ring_body, 0, unroll=False)

    for j in range(NCHUNKS - LAG, NCHUNKS):
        outcopy(j, j % NBUF).wait()


def kernel(x, table):
    x2d = x.reshape(B // CHUNK, CHUNK).astype(jnp.int32)
    out = _gather_all(table, x2d)
    return out.reshape(4096, 200, D)
